# Initial kernel scaffold; baseline (speedup 1.0000x reference)
import jax
import jax.numpy as jnp
from jax.experimental import pallas as pl


def kernel(operations, related_items, materials, resources, need_for_resources, need_for_materials, precedences, self_W1, self_b1, self_W2, self_b2, self_W3, self_b3, items_W1, items_b1, items_W2, items_b2, items_W3, items_b3, materials_W1, materials_b1, materials_W2, materials_b2, materials_W3, materials_b3, resources_W1, resources_b1, resources_W2, resources_b2, resources_W3, resources_b3, pred_W1, pred_b1, pred_W2, pred_b2, pred_W3, pred_b3, succ_W1, succ_b1, succ_W2, succ_b2, succ_W3, succ_b3, comb_W1, comb_b1, comb_W2, comb_b2, comb_W3, comb_b3):
    raise NotImplementedError("write your pallas kernel here")


# speedup vs baseline: 1.0001x; 1.0001x over previous
import jax
import jax.numpy as jnp
from jax.experimental import pallas as pl
from jax.experimental.pallas import tpu as pltpu


def _elu(x):
    return jnp.where(x > 0, x, jnp.exp(jnp.minimum(x, 0.0)) - 1.0)


def _mlp(x, W1, b1, W2, b2, W3, b3):
    h = _elu(jnp.dot(x, W1, preferred_element_type=jnp.float32) + b1)
    h = _elu(jnp.dot(h, W2, preferred_element_type=jnp.float32) + b2)
    return jnp.dot(h, W3, preferred_element_type=jnp.float32) + b3


def _fused_mlp_body(n_ops, rows, *refs):
    (ops, items, aggm, aggr, aggp, aggs) = refs[:6]
    w = refs[6:48]
    out_ref = refs[48]

    def W(i):
        return [w[i * 6 + k][...] for k in range(6)]

    self_e = _mlp(ops[...], *W(0))
    item_e = _mlp(items[...], *W(1))
    m_e = _mlp(aggm[...], *W(2))
    r_e = _mlp(aggr[...], *W(3))
    p_e = _mlp(aggp[...], *W(4))
    s_e = _mlp(aggs[...], *W(5))

    cW1, cb1, cW2, cb2, cW3, cb3 = W(6)
    h = (jnp.dot(p_e, cW1[0:64], preferred_element_type=jnp.float32)
         + jnp.dot(s_e, cW1[64:128], preferred_element_type=jnp.float32)
         + jnp.dot(r_e, cW1[128:192], preferred_element_type=jnp.float32)
         + jnp.dot(m_e, cW1[192:256], preferred_element_type=jnp.float32)
         + jnp.dot(item_e, cW1[256:320], preferred_element_type=jnp.float32)
         + jnp.dot(self_e, cW1[320:384], preferred_element_type=jnp.float32)
         + cb1)
    h = _elu(h)
    h = _elu(jnp.dot(h, cW2, preferred_element_type=jnp.float32) + cb2)
    o = jnp.dot(h, cW3, preferred_element_type=jnp.float32) + cb3

    i = pl.program_id(0)
    gid = i * rows + jax.lax.broadcasted_iota(jnp.int32, (rows, 1), 0)
    mask = (gid >= 1) & (gid <= n_ops - 2)
    out_ref[...] = jnp.where(mask, o, 0.0)


def _fused_mlps(ops, items, aggm, aggr, aggp, aggs, weights):
    n_ops, _ = ops.shape
    rows = 1000 if n_ops % 1000 == 0 else n_ops
    grid = n_ops // rows

    data_spec = pl.BlockSpec((rows, 64), lambda i: (i, 0))
    w_specs = []
    w_in = []
    for (W1, b1, W2, b2, W3, b3) in weights:
        for arr in (W1, b1.reshape(1, -1), W2, b2.reshape(1, -1),
                    W3, b3.reshape(1, -1)):
            w_in.append(arr)
            w_specs.append(pl.BlockSpec(arr.shape, lambda i: (0, 0)))

    import functools
    body = functools.partial(_fused_mlp_body, n_ops, rows)
    return pl.pallas_call(
        body,
        grid=(grid,),
        in_specs=[data_spec] * 6 + w_specs,
        out_specs=pl.BlockSpec((rows, 64), lambda i: (i, 0)),
        out_shape=jax.ShapeDtypeStruct((n_ops, 64), jnp.float32),
        compiler_params=pltpu.CompilerParams(
            dimension_semantics=("parallel",)),
    )(ops, items, aggm, aggr, aggp, aggs, *w_in)


def kernel(operations, related_items, materials, resources, need_for_resources, need_for_materials, precedences, self_W1, self_b1, self_W2, self_b2, self_W3, self_b3, items_W1, items_b1, items_W2, items_b2, items_W3, items_b3, materials_W1, materials_b1, materials_W2, materials_b2, materials_W3, materials_b3, resources_W1, resources_b1, resources_W2, resources_b2, resources_W3, resources_b3, pred_W1, pred_b1, pred_W2, pred_b2, pred_W3, pred_b3, succ_W1, succ_b1, succ_W2, succ_b2, succ_W3, succ_b3, comb_W1, comb_b1, comb_W2, comb_b2, comb_W3, comb_b3):
    n_ops = operations.shape[0]

    agg_mat = jnp.zeros((n_ops, materials.shape[1]), jnp.float32).at[
        need_for_materials[0]].add(materials[need_for_materials[1]])
    agg_res = jnp.zeros((n_ops, resources.shape[1]), jnp.float32).at[
        need_for_resources[0]].add(resources[need_for_resources[1]])
    agg_pred = jnp.zeros((n_ops, operations.shape[1]), jnp.float32).at[
        precedences[0]].add(resources[precedences[1]])
    agg_succ = jnp.zeros((n_ops, operations.shape[1]), jnp.float32).at[
        precedences[1]].add(resources[precedences[0]])

    weights = [
        (self_W1, self_b1, self_W2, self_b2, self_W3, self_b3),
        (items_W1, items_b1, items_W2, items_b2, items_W3, items_b3),
        (materials_W1, materials_b1, materials_W2, materials_b2, materials_W3, materials_b3),
        (resources_W1, resources_b1, resources_W2, resources_b2, resources_W3, resources_b3),
        (pred_W1, pred_b1, pred_W2, pred_b2, pred_W3, pred_b3),
        (succ_W1, succ_b1, succ_W2, succ_b2, succ_W3, succ_b3),
        (comb_W1, comb_b1, comb_W2, comb_b2, comb_W3, comb_b3),
    ]
    return _fused_mlps(operations, related_items, agg_mat, agg_res,
                       agg_pred, agg_succ, weights)
